# baseline (speedup 1.0000x reference)
import functools

import jax
import jax.numpy as jnp
from jax import lax
from jax.experimental import pallas as pl
from jax.experimental.pallas import tpu as pltpu
from jax.experimental.pallas import tpu_sc as plsc

EPS = 1e-5
LAM = 0.1
LN2 = 0.6931471805599453

NC = 2
NS = 16
NW = NC * NS
L = 16

NB = 8
NCH = 256
NH = 128
WDIM = 128

HSC = 32
HPW = HSC // 4
PPW = HPW * WDIM
CHC = 8
NCHUNK = NCH // CHC
NVEC = PPW // L
VPH = WDIM // L

HTC = NH - HSC
HB = 16
NHB = HTC // HB
NSTEP = NB * NHB


def _rsqrt(q):
    bits = lax.bitcast_convert_type(q, jnp.int32)
    y = lax.bitcast_convert_type(jnp.int32(0x5F3759DF) - (bits >> 1),
                                 jnp.float32)
    for _ in range(3):
        y = y * (1.5 - 0.5 * q * y * y)
    return y


def _log(v):
    bits = lax.bitcast_convert_type(v, jnp.int32)
    e = ((bits >> 23) - 127).astype(jnp.float32)
    m = lax.bitcast_convert_type((bits & 0x007FFFFF) | 0x3F800000,
                                 jnp.float32)
    s = (m - 1.0) / (m + 1.0)
    z = s * s
    p = s * (2.0 + z * (0.66666667 + z * (0.4 + z * (0.28571429
                                                     + z * 0.22222222))))
    return e * jnp.float32(LN2) + p


def _sc_body(x_hbm, t_hbm, pt_hbm, out_hbm,
             ptbuf, tbuf, tcb, pn2buf, xbuf, assq, apd, obuf, sem):
    c = lax.axis_index("c")
    s = lax.axis_index("s")
    wid = s * NC + c
    b = wid // 4
    h0 = (wid % 4) * HPW

    pltpu.sync_copy(pt_hbm, ptbuf)
    pltpu.sync_copy(t_hbm.at[b, pl.ds(h0, HPW), :], tbuf)

    zeros = jnp.zeros((L,), jnp.float32)
    lanes = lax.iota(jnp.int32, L)

    @plsc.parallel_loop(0, NCH, unroll=4, carry=(zeros,) * 7)
    def pn2_acc(ch, accs):
        return tuple(
            acc + pv * pv
            for acc, pv in (
                (accs[cv],
                 plsc.load_gather(ptbuf, [lanes + (cv * L + ch * 100)]))
                for cv in range(7)))

    for cv in range(7):
        pn2buf[pl.ds(cv * L, L)] = pn2_acc[cv]

    def tprep(pv, _):
        h = pv // VPH
        wo = (pv % VPH) * L
        t16 = tbuf[h, pl.ds(wo, L)]
        tcb[pl.ds(pv * L, L)] = jnp.maximum(jnp.minimum(t16, 99), 0)
        return 0

    lax.fori_loop(0, NVEC, tprep, 0)

    def zloop(i, _):
        o = i * L
        assq[pl.ds(o, L)] = zeros
        apd[pl.ds(o, L)] = zeros
        return 0

    lax.fori_loop(0, NVEC, zloop, 0)

    def _src(cc):
        return x_hbm.at[b, pl.ds(cc * CHC, CHC), pl.ds(h0, HPW), :]

    def _process(cc, slot):
        cbase = cc * (CHC * 100)

        @plsc.parallel_loop(0, NVEC, unroll=4)
        def pvec(pv):
            o = pv * L
            h = pv // VPH
            wo = (pv % VPH) * L
            base = tcb[pl.ds(o, L)] + cbase
            sacc = zeros
            pacc = zeros
            for k in range(CHC):
                xv = xbuf[slot, k, h, pl.ds(wo, L)]
                pvals = plsc.load_gather(ptbuf, [base + k * 100])
                sacc = sacc + xv * xv
                pacc = pacc + pvals * xv
            assq[pl.ds(o, L)] += sacc
            apd[pl.ds(o, L)] += pacc

    copies = [None] * NCHUNK
    copies[0] = pltpu.async_copy(_src(0), xbuf.at[0], sem)
    for cc in range(NCHUNK):
        slot = cc & 1
        copies[cc].wait()
        if cc + 1 < NCHUNK:
            copies[cc + 1] = pltpu.async_copy(_src(cc + 1), xbuf.at[1 - slot],
                                              sem)
        _process(cc, slot)

    @plsc.parallel_loop(0, NVEC, unroll=2, carry=(zeros, zeros))
    def epilogue(pv, carry):
        sv_acc, sm_acc = carry
        o = pv * L
        h = pv // VPH
        wo = (pv % VPH) * L
        q = assq[pl.ds(o, L)]
        pd = apd[pl.ds(o, L)]
        t16 = tbuf[h, pl.ds(wo, L)]
        pn = plsc.load_gather(pn2buf, [tcb[pl.ds(o, L)]])
        r = jnp.maximum(q * _rsqrt(q), 1e-15)
        th = 1.0 - 2.0 / (jnp.exp(2.0 * r) + 1.0)
        scale = th / r
        nx = th * th
        denom = jnp.maximum(1.0 - nx, EPS)
        sq = pn + nx - 2.0 * (scale * pd)
        val = _log(jnp.maximum(sq / denom, EPS)) - LAM * _log(denom)
        m = ((t16 != 255) & (t16 != -1)).astype(jnp.float32)
        return sv_acc + val * m, sm_acc + m

    sv, sm = epilogue
    obuf[0, :] = sv
    obuf[1, :] = sm
    pltpu.sync_copy(obuf, out_hbm.at[wid])


def _tc_body(xref, tref, pref, oref, acc):
    g = pl.program_id(0)
    X = xref[0].reshape(256, HB * 128)
    P = pref[...]
    t = tref[0].reshape(1, HB * 128)

    ssq = jnp.sum(X * X, axis=0, keepdims=True)
    S = jax.lax.dot_general(P, X, (((1,), (0,)), ((), ())),
                            preferred_element_type=jnp.float32)
    pn2 = jnp.sum(P * P, axis=1, keepdims=True)

    iot = jax.lax.broadcasted_iota(jnp.int32, (100, 1), 0)
    O = t == iot
    dsel = jnp.sum(jnp.where(O, S, 0.0), axis=0, keepdims=True)
    pn2sel = jnp.sum(jnp.where(O, jnp.broadcast_to(pn2, O.shape), 0.0),
                     axis=0, keepdims=True)

    r = jnp.maximum(jnp.sqrt(ssq), 1e-15)
    th = jnp.tanh(r)
    scale = th / r
    nx = th * th
    denom = jnp.maximum(1.0 - nx, EPS)
    sq = pn2sel + nx - 2.0 * (scale * dsel)
    val = jnp.log(jnp.maximum(sq / denom, EPS)) - LAM * jnp.log(denom)
    m = ((t != 255) & (t != -1)).astype(jnp.float32)

    @pl.when(g == 0)
    def _init():
        acc[0] = 0.0
        acc[1] = 0.0

    acc[0] += jnp.sum(val * m)
    acc[1] += jnp.sum(m)

    @pl.when(g == NSTEP - 1)
    def _fin():
        oref[0, 0] = acc[0]
        oref[0, 1] = acc[1]


def _reduce_body(scref, tcref, oref):
    sv = jnp.sum(scref[:, 0, :]) + tcref[0, 0]
    sm = jnp.sum(scref[:, 1, :]) + tcref[0, 1]
    oref[0, 0] = sv / sm


@functools.partial(jax.jit, static_argnums=())
def kernel(x, targets, protos):
    ptflat = jnp.transpose(protos).reshape(NCH * 100)

    mesh = plsc.VectorSubcoreMesh(core_axis_name="c", subcore_axis_name="s")
    sc_parts = pl.kernel(
        _sc_body,
        out_type=jax.ShapeDtypeStruct((NW, 2, L), jnp.float32),
        mesh=mesh,
        compiler_params=pltpu.CompilerParams(needs_layout_passes=False),
        scratch_types=[
            pltpu.VMEM((NCH * 100,), jnp.float32),
            pltpu.VMEM((HPW, WDIM), jnp.int32),
            pltpu.VMEM((PPW,), jnp.int32),
            pltpu.VMEM((7 * L,), jnp.float32),
            pltpu.VMEM((2, CHC, HPW, WDIM), jnp.float32),
            pltpu.VMEM((PPW,), jnp.float32),
            pltpu.VMEM((PPW,), jnp.float32),
            pltpu.VMEM((2, L), jnp.float32),
            pltpu.SemaphoreType.DMA,
        ],
    )(x, targets, ptflat)

    tc_parts = pl.pallas_call(
        _tc_body,
        grid=(NSTEP,),
        in_specs=[
            pl.BlockSpec((1, 256, HB, 128),
                         lambda g: (g // NHB, 0, HSC // HB + g % NHB, 0)),
            pl.BlockSpec((1, HB, 128),
                         lambda g: (g // NHB, HSC // HB + g % NHB, 0)),
            pl.BlockSpec((100, 256), lambda g: (0, 0)),
        ],
        out_specs=pl.BlockSpec(memory_space=pltpu.SMEM),
        out_shape=jax.ShapeDtypeStruct((1, 2), jnp.float32),
        scratch_shapes=[pltpu.SMEM((2,), jnp.float32)],
    )(x, targets, protos)

    sv = jnp.sum(sc_parts[:, 0, :]) + tc_parts[0, 0]
    sm = jnp.sum(sc_parts[:, 1, :]) + tc_parts[0, 1]
    return sv / sm


# speedup vs baseline: 1.0473x; 1.0473x over previous
import functools

import jax
import jax.numpy as jnp
from jax import lax
from jax.experimental import pallas as pl
from jax.experimental.pallas import tpu as pltpu
from jax.experimental.pallas import tpu_sc as plsc

EPS = 1e-5
LAM = 0.1
LN2 = 0.6931471805599453

NC = 2
NS = 16
NW = NC * NS
L = 16

NB = 8
NCH = 256
NH = 128
WDIM = 128

HSC = 32
HPW = HSC // 4
PPW = HPW * WDIM
CHC = 8
NCHUNK = NCH // CHC
NVEC = PPW // L
VPH = WDIM // L

HTC = NH - HSC
HB = 16
NHB = HTC // HB
NSTEP = NB * NHB


def _rsqrt(q):
    bits = lax.bitcast_convert_type(q, jnp.int32)
    y = lax.bitcast_convert_type(jnp.int32(0x5F3759DF) - (bits >> 1),
                                 jnp.float32)
    for _ in range(3):
        y = y * (1.5 - 0.5 * q * y * y)
    return y


def _log(v):
    bits = lax.bitcast_convert_type(v, jnp.int32)
    e = ((bits >> 23) - 127).astype(jnp.float32)
    m = lax.bitcast_convert_type((bits & 0x007FFFFF) | 0x3F800000,
                                 jnp.float32)
    s = (m - 1.0) / (m + 1.0)
    z = s * s
    p = s * (2.0 + z * (0.66666667 + z * (0.4 + z * (0.28571429
                                                     + z * 0.22222222))))
    return e * jnp.float32(LN2) + p


def _sc_body(x_hbm, t_hbm, pt_hbm, out_hbm,
             ptbuf, tbuf, tcb, pn2buf, xbuf, assq, apd, obuf, sem):
    c = lax.axis_index("c")
    s = lax.axis_index("s")
    wid = s * NC + c
    b = wid // 4
    h0 = (wid % 4) * HPW

    pltpu.sync_copy(pt_hbm, ptbuf)
    pltpu.sync_copy(t_hbm.at[b, pl.ds(h0, HPW), :], tbuf)

    zeros = jnp.zeros((L,), jnp.float32)
    lanes = lax.iota(jnp.int32, L)

    @plsc.parallel_loop(0, NCH, unroll=4, carry=(zeros,) * 7)
    def pn2_acc(ch, accs):
        return tuple(
            acc + pv * pv
            for acc, pv in (
                (accs[cv],
                 plsc.load_gather(ptbuf, [lanes + (cv * L + ch * 100)]))
                for cv in range(7)))

    for cv in range(7):
        pn2buf[pl.ds(cv * L, L)] = pn2_acc[cv]

    def tprep(pv, _):
        h = pv // VPH
        wo = (pv % VPH) * L
        t16 = tbuf[h, pl.ds(wo, L)]
        tcb[pl.ds(pv * L, L)] = jnp.maximum(jnp.minimum(t16, 99), 0)
        return 0

    lax.fori_loop(0, NVEC, tprep, 0)

    def zloop(i, _):
        o = i * L
        assq[pl.ds(o, L)] = zeros
        apd[pl.ds(o, L)] = zeros
        return 0

    lax.fori_loop(0, NVEC, zloop, 0)

    def _src(cc):
        return x_hbm.at[b, pl.ds(cc * CHC, CHC), pl.ds(h0, HPW), :]

    def _process(cc, slot):
        cbase = cc * (CHC * 100)

        @plsc.parallel_loop(0, NVEC, unroll=4)
        def pvec(pv):
            o = pv * L
            h = pv // VPH
            wo = (pv % VPH) * L
            base = tcb[pl.ds(o, L)] + cbase
            sacc = zeros
            pacc = zeros
            for k in range(CHC):
                xv = xbuf[slot, k, h, pl.ds(wo, L)]
                pvals = plsc.load_gather(ptbuf, [base + k * 100])
                sacc = sacc + xv * xv
                pacc = pacc + pvals * xv
            assq[pl.ds(o, L)] += sacc
            apd[pl.ds(o, L)] += pacc

    pltpu.async_copy(_src(0), xbuf.at[0], sem)
    pltpu.async_copy(_src(1), xbuf.at[1], sem)

    def chunk_pair(i, _):
        cc = i * 2
        for s in range(2):
            pltpu.make_async_copy(_src(cc + s), xbuf.at[s], sem).wait()
            _process(cc + s, s)

            @pl.when(cc + s + 2 < NCHUNK)
            def _prefetch():
                pltpu.async_copy(_src(cc + s + 2), xbuf.at[s], sem)

        return 0

    lax.fori_loop(0, NCHUNK // 2, chunk_pair, 0)

    @plsc.parallel_loop(0, NVEC, unroll=2, carry=(zeros, zeros))
    def epilogue(pv, carry):
        sv_acc, sm_acc = carry
        o = pv * L
        h = pv // VPH
        wo = (pv % VPH) * L
        q = assq[pl.ds(o, L)]
        pd = apd[pl.ds(o, L)]
        t16 = tbuf[h, pl.ds(wo, L)]
        pn = plsc.load_gather(pn2buf, [tcb[pl.ds(o, L)]])
        r = jnp.maximum(q * _rsqrt(q), 1e-15)
        th = 1.0 - 2.0 / (jnp.exp(2.0 * r) + 1.0)
        scale = th / r
        nx = th * th
        denom = jnp.maximum(1.0 - nx, EPS)
        sq = pn + nx - 2.0 * (scale * pd)
        val = _log(jnp.maximum(sq / denom, EPS)) - LAM * _log(denom)
        m = ((t16 != 255) & (t16 != -1)).astype(jnp.float32)
        return sv_acc + val * m, sm_acc + m

    sv, sm = epilogue
    obuf[0, :] = sv
    obuf[1, :] = sm
    pltpu.sync_copy(obuf, out_hbm.at[wid])


def _tc_body(xref, tref, pref, oref, acc):
    g = pl.program_id(0)
    X = xref[0].reshape(256, HB * 128)
    P = pref[...]
    t = tref[0].reshape(1, HB * 128)

    ssq = jnp.sum(X * X, axis=0, keepdims=True)
    S = jax.lax.dot_general(P, X, (((1,), (0,)), ((), ())),
                            preferred_element_type=jnp.float32)
    pn2 = jnp.sum(P * P, axis=1, keepdims=True)

    iot = jax.lax.broadcasted_iota(jnp.int32, (100, 1), 0)
    O = t == iot
    dsel = jnp.sum(jnp.where(O, S, 0.0), axis=0, keepdims=True)
    pn2sel = jnp.sum(jnp.where(O, jnp.broadcast_to(pn2, O.shape), 0.0),
                     axis=0, keepdims=True)

    r = jnp.maximum(jnp.sqrt(ssq), 1e-15)
    th = jnp.tanh(r)
    scale = th / r
    nx = th * th
    denom = jnp.maximum(1.0 - nx, EPS)
    sq = pn2sel + nx - 2.0 * (scale * dsel)
    val = jnp.log(jnp.maximum(sq / denom, EPS)) - LAM * jnp.log(denom)
    m = ((t != 255) & (t != -1)).astype(jnp.float32)

    @pl.when(g == 0)
    def _init():
        acc[0] = 0.0
        acc[1] = 0.0

    acc[0] += jnp.sum(val * m)
    acc[1] += jnp.sum(m)

    @pl.when(g == NSTEP - 1)
    def _fin():
        oref[0, 0] = acc[0]
        oref[0, 1] = acc[1]


def _reduce_body(scref, tcref, oref):
    sv = jnp.sum(scref[:, 0, :]) + tcref[0, 0]
    sm = jnp.sum(scref[:, 1, :]) + tcref[0, 1]
    oref[0, 0] = sv / sm


@functools.partial(jax.jit, static_argnums=())
def kernel(x, targets, protos):
    ptflat = jnp.transpose(protos).reshape(NCH * 100)

    mesh = plsc.VectorSubcoreMesh(core_axis_name="c", subcore_axis_name="s")
    sc_parts = pl.kernel(
        _sc_body,
        out_type=jax.ShapeDtypeStruct((NW, 2, L), jnp.float32),
        mesh=mesh,
        compiler_params=pltpu.CompilerParams(needs_layout_passes=False),
        scratch_types=[
            pltpu.VMEM((NCH * 100,), jnp.float32),
            pltpu.VMEM((HPW, WDIM), jnp.int32),
            pltpu.VMEM((PPW,), jnp.int32),
            pltpu.VMEM((7 * L,), jnp.float32),
            pltpu.VMEM((2, CHC, HPW, WDIM), jnp.float32),
            pltpu.VMEM((PPW,), jnp.float32),
            pltpu.VMEM((PPW,), jnp.float32),
            pltpu.VMEM((2, L), jnp.float32),
            pltpu.SemaphoreType.DMA,
        ],
    )(x, targets, ptflat)

    tc_parts = pl.pallas_call(
        _tc_body,
        grid=(NSTEP,),
        in_specs=[
            pl.BlockSpec((1, 256, HB, 128),
                         lambda g: (g // NHB, 0, HSC // HB + g % NHB, 0)),
            pl.BlockSpec((1, HB, 128),
                         lambda g: (g // NHB, HSC // HB + g % NHB, 0)),
            pl.BlockSpec((100, 256), lambda g: (0, 0)),
        ],
        out_specs=pl.BlockSpec(memory_space=pltpu.SMEM),
        out_shape=jax.ShapeDtypeStruct((1, 2), jnp.float32),
        scratch_shapes=[pltpu.SMEM((2,), jnp.float32)],
    )(x, targets, protos)

    out = pl.pallas_call(
        _reduce_body,
        in_specs=[
            pl.BlockSpec((NW, 2, L), lambda: (0, 0, 0)),
            pl.BlockSpec(memory_space=pltpu.SMEM),
        ],
        out_specs=pl.BlockSpec(memory_space=pltpu.SMEM),
        out_shape=jax.ShapeDtypeStruct((1, 1), jnp.float32),
    )(sc_parts, tc_parts)
    return out[0, 0]


# speedup vs baseline: 1.0696x; 1.0213x over previous
import functools

import jax
import jax.numpy as jnp
from jax import lax
from jax.experimental import pallas as pl
from jax.experimental.pallas import tpu as pltpu
from jax.experimental.pallas import tpu_sc as plsc

EPS = 1e-5
LAM = 0.1
LN2 = 0.6931471805599453

NC = 2
NS = 16
NW = NC * NS
L = 16

NB = 8
NCH = 256
NH = 128
WDIM = 128

HSC = 64
HPW = HSC // 4
PPW = HPW * WDIM
CHC = 8
NCHUNK = NCH // CHC
NVEC = PPW // L
VPH = WDIM // L

HTC = NH - HSC
HB = 16
NHB = HTC // HB
NSTEP = NB * NHB


def _rsqrt(q):
    bits = lax.bitcast_convert_type(q, jnp.int32)
    y = lax.bitcast_convert_type(jnp.int32(0x5F3759DF) - (bits >> 1),
                                 jnp.float32)
    for _ in range(3):
        y = y * (1.5 - 0.5 * q * y * y)
    return y


def _log(v):
    bits = lax.bitcast_convert_type(v, jnp.int32)
    e = ((bits >> 23) - 127).astype(jnp.float32)
    m = lax.bitcast_convert_type((bits & 0x007FFFFF) | 0x3F800000,
                                 jnp.float32)
    s = (m - 1.0) / (m + 1.0)
    z = s * s
    p = s * (2.0 + z * (0.66666667 + z * (0.4 + z * (0.28571429
                                                     + z * 0.22222222))))
    return e * jnp.float32(LN2) + p


def _sc_body(x_hbm, t_hbm, pt_hbm, out_hbm,
             ptbuf, tbuf, tcb, pn2buf, xbuf, assq, apd, obuf, sem):
    c = lax.axis_index("c")
    s = lax.axis_index("s")
    wid = s * NC + c
    b = wid // 4
    h0 = (wid % 4) * HPW

    pltpu.sync_copy(pt_hbm, ptbuf)
    pltpu.sync_copy(t_hbm.at[b, pl.ds(h0, HPW), :], tbuf)

    zeros = jnp.zeros((L,), jnp.float32)
    lanes = lax.iota(jnp.int32, L)

    @plsc.parallel_loop(0, NCH, unroll=4, carry=(zeros,) * 7)
    def pn2_acc(ch, accs):
        return tuple(
            acc + pv * pv
            for acc, pv in (
                (accs[cv],
                 plsc.load_gather(ptbuf, [lanes + (cv * L + ch * 100)]))
                for cv in range(7)))

    for cv in range(7):
        pn2buf[pl.ds(cv * L, L)] = pn2_acc[cv]

    def tprep(pv, _):
        h = pv // VPH
        wo = (pv % VPH) * L
        t16 = tbuf[h, pl.ds(wo, L)]
        tcb[pl.ds(pv * L, L)] = jnp.maximum(jnp.minimum(t16, 99), 0)
        return 0

    lax.fori_loop(0, NVEC, tprep, 0)

    def zloop(i, _):
        o = i * L
        assq[pl.ds(o, L)] = zeros
        apd[pl.ds(o, L)] = zeros
        return 0

    lax.fori_loop(0, NVEC, zloop, 0)

    def _src(cc):
        return x_hbm.at[b, pl.ds(cc * CHC, CHC), pl.ds(h0, HPW), :]

    def _process(cc, slot):
        cbase = cc * (CHC * 100)

        @plsc.parallel_loop(0, NVEC, unroll=4)
        def pvec(pv):
            o = pv * L
            h = pv // VPH
            wo = (pv % VPH) * L
            base = tcb[pl.ds(o, L)] + cbase
            sacc = zeros
            pacc = zeros
            for k in range(CHC):
                xv = xbuf[slot, k, h, pl.ds(wo, L)]
                pvals = plsc.load_gather(ptbuf, [base + k * 100])
                sacc = sacc + xv * xv
                pacc = pacc + pvals * xv
            assq[pl.ds(o, L)] += sacc
            apd[pl.ds(o, L)] += pacc

    pltpu.async_copy(_src(0), xbuf.at[0], sem)
    pltpu.async_copy(_src(1), xbuf.at[1], sem)

    def chunk_pair(i, _):
        cc = i * 2
        for s in range(2):
            pltpu.make_async_copy(_src(cc + s), xbuf.at[s], sem).wait()
            _process(cc + s, s)

            @pl.when(cc + s + 2 < NCHUNK)
            def _prefetch():
                pltpu.async_copy(_src(cc + s + 2), xbuf.at[s], sem)

        return 0

    lax.fori_loop(0, NCHUNK // 2, chunk_pair, 0)

    @plsc.parallel_loop(0, NVEC, unroll=2, carry=(zeros, zeros))
    def epilogue(pv, carry):
        sv_acc, sm_acc = carry
        o = pv * L
        h = pv // VPH
        wo = (pv % VPH) * L
        q = assq[pl.ds(o, L)]
        pd = apd[pl.ds(o, L)]
        t16 = tbuf[h, pl.ds(wo, L)]
        pn = plsc.load_gather(pn2buf, [tcb[pl.ds(o, L)]])
        r = jnp.maximum(q * _rsqrt(q), 1e-15)
        th = 1.0 - 2.0 / (jnp.exp(2.0 * r) + 1.0)
        scale = th / r
        nx = th * th
        denom = jnp.maximum(1.0 - nx, EPS)
        sq = pn + nx - 2.0 * (scale * pd)
        val = _log(jnp.maximum(sq / denom, EPS)) - LAM * _log(denom)
        m = ((t16 != 255) & (t16 != -1)).astype(jnp.float32)
        return sv_acc + val * m, sm_acc + m

    sv, sm = epilogue
    obuf[0, :] = sv
    obuf[1, :] = sm
    pltpu.sync_copy(obuf, out_hbm.at[wid])


def _tc_body(xref, tref, pref, oref, acc):
    g = pl.program_id(0)
    X = xref[0].reshape(256, HB * 128)
    P = pref[...]
    t = tref[0].reshape(1, HB * 128)

    ssq = jnp.sum(X * X, axis=0, keepdims=True)
    S = jax.lax.dot_general(P, X, (((1,), (0,)), ((), ())),
                            preferred_element_type=jnp.float32)
    pn2 = jnp.sum(P * P, axis=1, keepdims=True)

    iot = jax.lax.broadcasted_iota(jnp.int32, (100, 1), 0)
    O = t == iot
    dsel = jnp.sum(jnp.where(O, S, 0.0), axis=0, keepdims=True)
    pn2sel = jnp.sum(jnp.where(O, jnp.broadcast_to(pn2, O.shape), 0.0),
                     axis=0, keepdims=True)

    r = jnp.maximum(jnp.sqrt(ssq), 1e-15)
    th = jnp.tanh(r)
    scale = th / r
    nx = th * th
    denom = jnp.maximum(1.0 - nx, EPS)
    sq = pn2sel + nx - 2.0 * (scale * dsel)
    val = jnp.log(jnp.maximum(sq / denom, EPS)) - LAM * jnp.log(denom)
    m = ((t != 255) & (t != -1)).astype(jnp.float32)

    @pl.when(g == 0)
    def _init():
        acc[0] = 0.0
        acc[1] = 0.0

    acc[0] += jnp.sum(val * m)
    acc[1] += jnp.sum(m)

    @pl.when(g == NSTEP - 1)
    def _fin():
        oref[0, 0] = acc[0]
        oref[0, 1] = acc[1]


def _reduce_body(scref, tcref, oref):
    sv = jnp.sum(scref[:, 0, :]) + tcref[0, 0]
    sm = jnp.sum(scref[:, 1, :]) + tcref[0, 1]
    oref[0, 0] = sv / sm


@functools.partial(jax.jit, static_argnums=())
def kernel(x, targets, protos):
    ptflat = jnp.transpose(protos).reshape(NCH * 100)

    mesh = plsc.VectorSubcoreMesh(core_axis_name="c", subcore_axis_name="s")
    sc_parts = pl.kernel(
        _sc_body,
        out_type=jax.ShapeDtypeStruct((NW, 2, L), jnp.float32),
        mesh=mesh,
        compiler_params=pltpu.CompilerParams(needs_layout_passes=False),
        scratch_types=[
            pltpu.VMEM((NCH * 100,), jnp.float32),
            pltpu.VMEM((HPW, WDIM), jnp.int32),
            pltpu.VMEM((PPW,), jnp.int32),
            pltpu.VMEM((7 * L,), jnp.float32),
            pltpu.VMEM((2, CHC, HPW, WDIM), jnp.float32),
            pltpu.VMEM((PPW,), jnp.float32),
            pltpu.VMEM((PPW,), jnp.float32),
            pltpu.VMEM((2, L), jnp.float32),
            pltpu.SemaphoreType.DMA,
        ],
    )(x, targets, ptflat)

    tc_parts = pl.pallas_call(
        _tc_body,
        grid=(NSTEP,),
        in_specs=[
            pl.BlockSpec((1, 256, HB, 128),
                         lambda g: (g // NHB, 0, HSC // HB + g % NHB, 0)),
            pl.BlockSpec((1, HB, 128),
                         lambda g: (g // NHB, HSC // HB + g % NHB, 0)),
            pl.BlockSpec((100, 256), lambda g: (0, 0)),
        ],
        out_specs=pl.BlockSpec(memory_space=pltpu.SMEM),
        out_shape=jax.ShapeDtypeStruct((1, 2), jnp.float32),
        scratch_shapes=[pltpu.SMEM((2,), jnp.float32)],
    )(x, targets, protos)

    out = pl.pallas_call(
        _reduce_body,
        in_specs=[
            pl.BlockSpec((NW, 2, L), lambda: (0, 0, 0)),
            pl.BlockSpec(memory_space=pltpu.SMEM),
        ],
        out_specs=pl.BlockSpec(memory_space=pltpu.SMEM),
        out_shape=jax.ShapeDtypeStruct((1, 1), jnp.float32),
    )(sc_parts, tc_parts)
    return out[0, 0]


# speedup vs baseline: 1.1272x; 1.0538x over previous
import functools

import jax
import jax.numpy as jnp
from jax import lax
from jax.experimental import pallas as pl
from jax.experimental.pallas import tpu as pltpu
from jax.experimental.pallas import tpu_sc as plsc

EPS = 1e-5
LAM = 0.1
LN2 = 0.6931471805599453

NC = 2
NS = 16
NW = NC * NS
L = 16

NB = 8
NCH = 256
NH = 128
WDIM = 128

HSC = 64
HPW = HSC // 4
PPW = HPW * WDIM
CHC = 8
NCHUNK = NCH // CHC
NVEC = PPW // L
VPH = WDIM // L

HTC = NH - HSC
HB = 16
NHB = HTC // HB
NSTEP = NB * NHB


def _rsqrt(q):
    bits = lax.bitcast_convert_type(q, jnp.int32)
    y = lax.bitcast_convert_type(jnp.int32(0x5F3759DF) - (bits >> 1),
                                 jnp.float32)
    for _ in range(3):
        y = y * (1.5 - 0.5 * q * y * y)
    return y


def _log(v):
    bits = lax.bitcast_convert_type(v, jnp.int32)
    e = ((bits >> 23) - 127).astype(jnp.float32)
    m = lax.bitcast_convert_type((bits & 0x007FFFFF) | 0x3F800000,
                                 jnp.float32)
    s = (m - 1.0) / (m + 1.0)
    z = s * s
    p = s * (2.0 + z * (0.66666667 + z * (0.4 + z * (0.28571429
                                                     + z * 0.22222222))))
    return e * jnp.float32(LN2) + p


def _sc_body(x_hbm, t_hbm, pt_hbm, pk_hbm, out_hbm,
             ptbuf, pkbuf, tbuf, tcb, pn2buf, xbuf, assq, apd, obuf, sem):
    c = lax.axis_index("c")
    s = lax.axis_index("s")
    wid = s * NC + c
    b = wid // 4
    h0 = (wid % 4) * HPW

    pltpu.sync_copy(pt_hbm, ptbuf)
    pltpu.sync_copy(pk_hbm, pkbuf)
    pltpu.sync_copy(t_hbm.at[b, pl.ds(h0, HPW), :], tbuf)

    zeros = jnp.zeros((L,), jnp.float32)
    lanes = lax.iota(jnp.int32, L)

    @plsc.parallel_loop(0, NCH, unroll=4, carry=(zeros,) * 7)
    def pn2_acc(ch, accs):
        return tuple(
            acc + pv * pv
            for acc, pv in (
                (accs[cv],
                 plsc.load_gather(ptbuf, [lanes + (cv * L + ch * 100)]))
                for cv in range(7)))

    for cv in range(7):
        pn2buf[pl.ds(cv * L, L)] = pn2_acc[cv]

    def tprep(pv, _):
        h = pv // VPH
        wo = (pv % VPH) * L
        t16 = tbuf[h, pl.ds(wo, L)]
        tcb[pl.ds(pv * L, L)] = jnp.maximum(jnp.minimum(t16, 99), 0)
        return 0

    lax.fori_loop(0, NVEC, tprep, 0)

    def zloop(i, _):
        o = i * L
        assq[pl.ds(o, L)] = zeros
        apd[pl.ds(o, L)] = zeros
        return 0

    lax.fori_loop(0, NVEC, zloop, 0)

    def _src(cc):
        return x_hbm.at[b, pl.ds(cc * CHC, CHC), pl.ds(h0, HPW), :]

    def _process(cc, slot):
        cb2 = cc * ((CHC // 2) * 100)

        @plsc.parallel_loop(0, NVEC, unroll=4)
        def pvec(pv):
            o = pv * L
            h = pv // VPH
            wo = (pv % VPH) * L
            base2 = tcb[pl.ds(o, L)] + cb2
            sacc = zeros
            pacc = zeros
            for k2 in range(CHC // 2):
                w = plsc.load_gather(pkbuf, [base2 + k2 * 100])
                x0 = xbuf[slot, 2 * k2, h, pl.ds(wo, L)]
                x1 = xbuf[slot, 2 * k2 + 1, h, pl.ds(wo, L)]
                pe = lax.bitcast_convert_type(w << 16, jnp.float32)
                po = lax.bitcast_convert_type(w & jnp.int32(-65536),
                                              jnp.float32)
                sacc = sacc + (x0 * x0 + x1 * x1)
                pacc = pacc + (pe * x0 + po * x1)
            assq[pl.ds(o, L)] += sacc
            apd[pl.ds(o, L)] += pacc

    pltpu.async_copy(_src(0), xbuf.at[0], sem)
    pltpu.async_copy(_src(1), xbuf.at[1], sem)

    def chunk_pair(i, _):
        cc = i * 2
        for s in range(2):
            pltpu.make_async_copy(_src(cc + s), xbuf.at[s], sem).wait()
            _process(cc + s, s)

            @pl.when(cc + s + 2 < NCHUNK)
            def _prefetch():
                pltpu.async_copy(_src(cc + s + 2), xbuf.at[s], sem)

        return 0

    lax.fori_loop(0, NCHUNK // 2, chunk_pair, 0)

    @plsc.parallel_loop(0, NVEC, unroll=2, carry=(zeros, zeros))
    def epilogue(pv, carry):
        sv_acc, sm_acc = carry
        o = pv * L
        h = pv // VPH
        wo = (pv % VPH) * L
        q = assq[pl.ds(o, L)]
        pd = apd[pl.ds(o, L)]
        t16 = tbuf[h, pl.ds(wo, L)]
        pn = plsc.load_gather(pn2buf, [tcb[pl.ds(o, L)]])
        r = jnp.maximum(q * _rsqrt(q), 1e-15)
        th = 1.0 - 2.0 / (jnp.exp(2.0 * r) + 1.0)
        scale = th / r
        nx = th * th
        denom = jnp.maximum(1.0 - nx, EPS)
        sq = pn + nx - 2.0 * (scale * pd)
        val = _log(jnp.maximum(sq / denom, EPS)) - LAM * _log(denom)
        m = ((t16 != 255) & (t16 != -1)).astype(jnp.float32)
        return sv_acc + val * m, sm_acc + m

    sv, sm = epilogue
    obuf[0, :] = sv
    obuf[1, :] = sm
    pltpu.sync_copy(obuf, out_hbm.at[wid])


def _tc_body(xref, tref, pref, oref, acc):
    g = pl.program_id(0)
    X = xref[0].reshape(256, HB * 128)
    P = pref[...]
    t = tref[0].reshape(1, HB * 128)

    ssq = jnp.sum(X * X, axis=0, keepdims=True)
    S = jax.lax.dot_general(P, X, (((1,), (0,)), ((), ())),
                            preferred_element_type=jnp.float32)
    pn2 = jnp.sum(P * P, axis=1, keepdims=True)

    iot = jax.lax.broadcasted_iota(jnp.int32, (100, 1), 0)
    O = t == iot
    dsel = jnp.sum(jnp.where(O, S, 0.0), axis=0, keepdims=True)
    pn2sel = jnp.sum(jnp.where(O, jnp.broadcast_to(pn2, O.shape), 0.0),
                     axis=0, keepdims=True)

    r = jnp.maximum(jnp.sqrt(ssq), 1e-15)
    th = jnp.tanh(r)
    scale = th / r
    nx = th * th
    denom = jnp.maximum(1.0 - nx, EPS)
    sq = pn2sel + nx - 2.0 * (scale * dsel)
    val = jnp.log(jnp.maximum(sq / denom, EPS)) - LAM * jnp.log(denom)
    m = ((t != 255) & (t != -1)).astype(jnp.float32)

    @pl.when(g == 0)
    def _init():
        acc[0] = 0.0
        acc[1] = 0.0

    acc[0] += jnp.sum(val * m)
    acc[1] += jnp.sum(m)

    @pl.when(g == NSTEP - 1)
    def _fin():
        oref[0, 0] = acc[0]
        oref[0, 1] = acc[1]


def _reduce_body(scref, tcref, oref):
    sv = jnp.sum(scref[:, 0, :]) + tcref[0, 0]
    sm = jnp.sum(scref[:, 1, :]) + tcref[0, 1]
    oref[0, 0] = sv / sm


@functools.partial(jax.jit, static_argnums=())
def kernel(x, targets, protos):
    pt = jnp.transpose(protos)
    ptflat = pt.reshape(NCH * 100)
    ptb = pt.astype(jnp.bfloat16)
    eb = lax.bitcast_convert_type(ptb[0::2], jnp.uint16).astype(jnp.int32)
    ob = lax.bitcast_convert_type(ptb[1::2], jnp.uint16).astype(jnp.int32)
    pkflat = (eb | (ob << 16)).reshape((NCH // 2) * 100)

    mesh = plsc.VectorSubcoreMesh(core_axis_name="c", subcore_axis_name="s")
    sc_parts = pl.kernel(
        _sc_body,
        out_type=jax.ShapeDtypeStruct((NW, 2, L), jnp.float32),
        mesh=mesh,
        compiler_params=pltpu.CompilerParams(needs_layout_passes=False),
        scratch_types=[
            pltpu.VMEM((NCH * 100,), jnp.float32),
            pltpu.VMEM(((NCH // 2) * 100,), jnp.int32),
            pltpu.VMEM((HPW, WDIM), jnp.int32),
            pltpu.VMEM((PPW,), jnp.int32),
            pltpu.VMEM((7 * L,), jnp.float32),
            pltpu.VMEM((2, CHC, HPW, WDIM), jnp.float32),
            pltpu.VMEM((PPW,), jnp.float32),
            pltpu.VMEM((PPW,), jnp.float32),
            pltpu.VMEM((2, L), jnp.float32),
            pltpu.SemaphoreType.DMA,
        ],
    )(x, targets, ptflat, pkflat)

    tc_parts = pl.pallas_call(
        _tc_body,
        grid=(NSTEP,),
        in_specs=[
            pl.BlockSpec((1, 256, HB, 128),
                         lambda g: (g // NHB, 0, HSC // HB + g % NHB, 0)),
            pl.BlockSpec((1, HB, 128),
                         lambda g: (g // NHB, HSC // HB + g % NHB, 0)),
            pl.BlockSpec((100, 256), lambda g: (0, 0)),
        ],
        out_specs=pl.BlockSpec(memory_space=pltpu.SMEM),
        out_shape=jax.ShapeDtypeStruct((1, 2), jnp.float32),
        scratch_shapes=[pltpu.SMEM((2,), jnp.float32)],
    )(x, targets, protos)

    out = pl.pallas_call(
        _reduce_body,
        in_specs=[
            pl.BlockSpec((NW, 2, L), lambda: (0, 0, 0)),
            pl.BlockSpec(memory_space=pltpu.SMEM),
        ],
        out_specs=pl.BlockSpec(memory_space=pltpu.SMEM),
        out_shape=jax.ShapeDtypeStruct((1, 1), jnp.float32),
    )(sc_parts, tc_parts)
    return out[0, 0]
